# baseline (speedup 1.0000x reference)
import functools

import jax
import jax.numpy as jnp
from jax import lax
from jax.experimental import pallas as pl
from jax.experimental.pallas import tpu as pltpu

T = 36
POOL = 3
CONV_KS = (10, 15)
MC = 144
bf16 = jnp.bfloat16


def _sigmoid(x):
    return 0.5 * (jnp.tanh(0.5 * x) + 1.0)


def _full(shape):
    nd = len(shape)
    return pl.BlockSpec(tuple(shape), lambda _n=nd: (0,) * _n)


def _mm576(lhs_ref, col0, mxu, lsr):
    for j, mc in enumerate(range(0, T * 16, MC)):
        chunk = lhs_ref[pl.ds(mc, MC), pl.ds(col0, 256)].astype(bf16)
        pltpu.matmul_acc_lhs(mc // 4, chunk, mxu,
                             load_staged_rhs=lsr if j == 0 else None)


def _pop576(out_ref, col0, mxu, bias):
    for mc in range(0, T * 16, MC):
        v = pltpu.matmul_pop(mc // 4, (MC, 256), jnp.float32, mxu)
        out_ref[pl.ds(mc, MC), pl.ds(col0, 256)] = v + bias


def _cell(g0, g1, g2, g3, c_prev):
    i = _sigmoid(g0)
    f = _sigmoid(g1)
    g = jnp.tanh(g2)
    o = _sigmoid(g3)
    c = f * c_prev + i * g
    return o * jnp.tanh(c), c


def _lstm_bidir_loop(xp_scr, whh16_scr, store_fwd, store_bwd, Bp, H):
    f32 = jnp.float32
    z = jnp.zeros((Bp, H), f32)
    z16 = jnp.zeros((Bp, H), bf16)

    def push(dirn, tile, msr, mxu):
        pltpu.matmul_push_rhs(
            whh16_scr[dirn, :, pl.ds((2 * mxu + tile) * 256, 256)], msr, mxu)

    for mxu in range(2):
        push(0, 0, 0, mxu)
        pltpu.matmul_acc_lhs(0, z16, mxu, load_staged_rhs=0)

    def gates_f(rf):
        xpf = xp_scr[pl.ds(rf, Bp), :]
        return (pltpu.matmul_pop(0, (Bp, 256), f32, 0) + xpf[:, 0:256],
                pltpu.matmul_pop(8, (Bp, 256), f32, 0) + xpf[:, 256:512],
                pltpu.matmul_pop(0, (Bp, 256), f32, 1) + xpf[:, 512:768],
                pltpu.matmul_pop(8, (Bp, 256), f32, 1) + xpf[:, 768:1024])

    def gates_b(rb):
        xpb = xp_scr[pl.ds(rb, Bp), :]
        return (pltpu.matmul_pop(16, (Bp, 256), f32, 0) + xpb[:, 1024:1280],
                pltpu.matmul_pop(24, (Bp, 256), f32, 0) + xpb[:, 1280:1536],
                pltpu.matmul_pop(16, (Bp, 256), f32, 1) + xpb[:, 1536:1792],
                pltpu.matmul_pop(24, (Bp, 256), f32, 1) + xpb[:, 1792:2048])

    def body(p2, carry):
        hf, cf, hb, cb = carry
        s0 = 2 * p2
        rf = pl.multiple_of(s0 * Bp, Bp)
        rb = pl.multiple_of((T - 1 - s0) * Bp, Bp)
        hf16 = hf.astype(bf16)
        hb16 = hb.astype(bf16)
        for mxu in range(2):
            pltpu.matmul_acc_lhs(0, hf16, mxu, load_staged_rhs=None)
            push(0, 1, 1, mxu)
            pltpu.matmul_acc_lhs(8, hf16, mxu, load_staged_rhs=1)
            push(1, 0, 0, mxu)
            pltpu.matmul_acc_lhs(16, hb16, mxu, load_staged_rhs=0)
            push(1, 1, 1, mxu)
            pltpu.matmul_acc_lhs(24, hb16, mxu, load_staged_rhs=1)
        hf, cf = _cell(*gates_f(rf), cf)
        store_fwd(rf, s0, hf)
        hb, cb = _cell(*gates_b(rb), cb)
        store_bwd(rb, T - 1 - s0, hb)
        s1 = s0 + 1
        rf = pl.multiple_of(s1 * Bp, Bp)
        rb = pl.multiple_of((T - 1 - s1) * Bp, Bp)
        hf16 = hf.astype(bf16)
        hb16 = hb.astype(bf16)
        for mxu in range(2):
            pltpu.matmul_acc_lhs(24, hb16, mxu, load_staged_rhs=None)
            push(1, 0, 0, mxu)
            pltpu.matmul_acc_lhs(16, hb16, mxu, load_staged_rhs=0)
            push(0, 1, 1, mxu)
            pltpu.matmul_acc_lhs(8, hf16, mxu, load_staged_rhs=1)
            push(0, 0, 0, mxu)
            pltpu.matmul_acc_lhs(0, hf16, mxu, load_staged_rhs=0)
        hb, cb = _cell(*gates_b(rb), cb)
        store_bwd(rb, T - 1 - s1, hb)
        hf, cf = _cell(*gates_f(rf), cf)
        store_fwd(rf, s1, hf)
        return hf, cf, hb, cb

    lax.fori_loop(0, T // 2, body, (z, z, z, z))


def _fused_kernel(sp_ref, cw_ref, cb_ref,
                  wih1_ref, b1_ref, whh1f_ref, whh1b_ref,
                  wih2f_ref, wih2b_ref, b2_ref, whh2f_ref, whh2b_ref,
                  fc1wf_hbm, fc1wb_hbm, fc1b_ref,
                  fc2w_ref, fc2b_ref, fc3w_ref, fc3b_ref,
                  o_ref,
                  pscr, cwscr, feat_scr, xp_scr, whh16_scr,
                  l1_scr, h2f_scr, h2b_scr, fc1wf_scr, fc1wb_scr,
                  semf, semb, *, Bp, H, FCH):
    f32 = jnp.float32
    CK2 = sp_ref.shape[1]
    K2 = CK2 // 4

    pltpu.make_async_copy(fc1wf_hbm, fc1wf_scr, semf).start()
    pltpu.make_async_copy(fc1wb_hbm, fc1wb_scr, semb).start()

    pscr[...] = jnp.zeros((T * Bp, 256), bf16)
    pscr[:, pl.ds(0, CK2)] = sp_ref[...]

    for p in range(POOL):
        cwscr[...] = jnp.zeros((256, 256), f32)
        r0 = 0
        for bi, K in enumerate(CONV_KS):
            off = p + (CONV_KS[-1] - 1) // 2 - (K - 1) // 2
            for c in range(4):
                rows = pl.ds(c * K2 + off, K)
                src = cw_ref[pl.ds(r0 + c * K, K), :]
                if bi == 0:
                    cwscr[rows, :] = src
                else:
                    cwscr[rows, :] = cwscr[rows, :] + src
            r0 += 4 * K
        mxu = p % 2
        pltpu.matmul_push_rhs(cwscr[...].astype(bf16), 0, mxu)
        _mm576(pscr, 0, mxu, 0)
        for mc in range(0, T * Bp, MC):
            v = pltpu.matmul_pop(mc // 4, (MC, 256), f32, mxu)
            if p == 0:
                feat_scr[pl.ds(mc, MC), :] = v
            elif p == 1:
                feat_scr[pl.ds(mc, MC), :] = jnp.maximum(
                    feat_scr[pl.ds(mc, MC), :], v)
            else:
                feat_scr[pl.ds(mc, MC), :] = jnp.maximum(
                    jnp.maximum(feat_scr[pl.ds(mc, MC), :], v) + cb_ref[...],
                    0.0)

    for n in range(8):
        mxu = n % 2
        pltpu.matmul_push_rhs(
            wih1_ref[:, pl.ds(n * 256, 256)].astype(bf16), 0, mxu)
        _mm576(feat_scr, 0, mxu, 0)
        _pop576(xp_scr, n * 256, mxu, b1_ref[0, pl.ds(n * 256, 256)][None, :])

    whh16_scr[0] = whh1f_ref[...].astype(bf16)
    whh16_scr[1] = whh1b_ref[...].astype(bf16)

    def store_fwd1(r, t, h):
        l1_scr[0, pl.ds(r, Bp), :] = h

    def store_bwd1(r, t, h):
        l1_scr[1, pl.ds(r, Bp), :] = h

    _lstm_bidir_loop(xp_scr, whh16_scr, store_fwd1, store_bwd1, Bp, H)

    for n in range(8):
        mxu = n % 2
        pltpu.matmul_push_rhs(
            wih2f_ref[:, pl.ds(n * 256, 256)].astype(bf16), 0, mxu)
        pltpu.matmul_push_rhs(
            wih2b_ref[:, pl.ds(n * 256, 256)].astype(bf16), 1, mxu)
        _mm576(l1_scr.at[0], 0, mxu, 0)
        _mm576(l1_scr.at[1], 0, mxu, 1)
        _pop576(xp_scr, n * 256, mxu, b2_ref[0, pl.ds(n * 256, 256)][None, :])

    whh16_scr[0] = whh2f_ref[...].astype(bf16)
    whh16_scr[1] = whh2b_ref[...].astype(bf16)

    def store_fwd(r, t, h):
        h2f_scr[:, pl.ds(pl.multiple_of(t * H, H), H)] = h

    def store_bwd(r, t, h):
        h2b_scr[:, pl.ds(pl.multiple_of(t * H, H), H)] = h

    _lstm_bidir_loop(xp_scr, whh16_scr, store_fwd, store_bwd, Bp, H)

    pltpu.make_async_copy(fc1wf_hbm, fc1wf_scr, semf).wait()
    pltpu.make_async_copy(fc1wb_hbm, fc1wb_scr, semb).wait()
    for kt in range(T):
        msr = kt % 2
        pltpu.matmul_push_rhs(
            fc1wf_scr[pl.ds(kt * 256, 256), :].astype(bf16), msr, 0)
        pltpu.matmul_acc_lhs(0, h2f_scr[:, pl.ds(kt * 256, 256)].astype(bf16),
                             0, load_staged_rhs=msr)
        pltpu.matmul_push_rhs(
            fc1wb_scr[pl.ds(kt * 256, 256), :].astype(bf16), msr, 1)
        pltpu.matmul_acc_lhs(0, h2b_scr[:, pl.ds(kt * 256, 256)].astype(bf16),
                             1, load_staged_rhs=msr)
    acc = (pltpu.matmul_pop(0, (Bp, FCH), f32, 0)
           + pltpu.matmul_pop(0, (Bp, FCH), f32, 1))

    y = jnp.maximum(acc + fc1b_ref[...], 0.0)
    pltpu.matmul_push_rhs(fc2w_ref[...].astype(bf16), 0, 0)
    pltpu.matmul_acc_lhs(0, y.astype(bf16), 0, load_staged_rhs=0)
    y = jnp.maximum(pltpu.matmul_pop(0, (Bp, FCH), f32, 0)
                    + fc2b_ref[...], 0.0)
    o_ref[...] = jnp.sum(y * fc3w_ref[...], axis=1, keepdims=True) + fc3b_ref[...]


def kernel(x, cw, cb, wih1, b1, whh1f, whh1b, wih2f, wih2b, b2, whh2f, whh2b,
           fc1wf, fc1wb, fc1b, fc2w, fc2b, fc3w, fc3b):
    f32 = jnp.float32
    B, L, Cin = x.shape
    H = whh1f.shape[0]
    FCH = fc2w.shape[0]
    C = cw.shape[1]
    Bp = max(8, (B + 7) // 8 * 8)

    xb = jnp.pad(x.astype(f32), ((0, Bp - B), (0, 0), (0, 0)))
    x_bcl = jnp.transpose(xb, (0, 2, 1))

    Kmax = max(CONV_KS)
    K2 = Kmax + POOL - 1
    pad_big = (Kmax - 1) // 2
    xpd = jnp.pad(x_bcl.astype(bf16),
                  ((0, 0), (0, 0), (pad_big, Kmax - 1 - pad_big)))
    idx = POOL * jnp.arange(T)[:, None] + jnp.arange(K2)[None, :]
    pt = xpd[:, :, idx]
    spatch = jnp.transpose(pt, (2, 0, 1, 3)).reshape(T * Bp, Cin * K2)
    CK2 = Cin * K2

    out = pl.pallas_call(
        functools.partial(_fused_kernel, Bp=Bp, H=H, FCH=FCH),
        out_shape=jax.ShapeDtypeStruct((Bp, 1), f32),
        in_specs=[
            _full((T * Bp, CK2)),
            _full((cw.shape[0], C)), _full((1, C)),
            _full((C, 8 * H)), _full((1, 8 * H)),
            _full((H, 4 * H)), _full((H, 4 * H)),
            _full((H, 8 * H)), _full((H, 8 * H)), _full((1, 8 * H)),
            _full((H, 4 * H)), _full((H, 4 * H)),
            pl.BlockSpec(memory_space=pl.ANY),
            pl.BlockSpec(memory_space=pl.ANY),
            _full((1, FCH)),
            _full((FCH, FCH)), _full((1, FCH)),
            _full((1, FCH)), _full((1, 1)),
        ],
        out_specs=_full((Bp, 1)),
        scratch_shapes=[
            pltpu.VMEM((T * Bp, 256), bf16),
            pltpu.VMEM((256, 256), f32),
            pltpu.VMEM((T * Bp, C), f32),
            pltpu.VMEM((T * Bp, 8 * H), f32),
            pltpu.VMEM((2, H, 4 * H), bf16),
            pltpu.VMEM((2, T * Bp, H), f32),
            pltpu.VMEM((Bp, T * H), f32),
            pltpu.VMEM((Bp, T * H), f32),
            pltpu.VMEM((T * H, FCH), f32),
            pltpu.VMEM((T * H, FCH), f32),
            pltpu.SemaphoreType.DMA,
            pltpu.SemaphoreType.DMA,
        ],
        grid=(),
    )(spatch, cw, cb, wih1, b1, whh1f, whh1b,
      wih2f, wih2b, b2, whh2f, whh2b, fc1wf, fc1wb, fc1b,
      fc2w, fc2b, fc3w, fc3b)

    return out[:B, 0]


# speedup vs baseline: 1.0234x; 1.0234x over previous
import functools

import jax
import jax.numpy as jnp
from jax import lax
from jax.experimental import pallas as pl
from jax.experimental.pallas import tpu as pltpu

T = 36
POOL = 3
CONV_KS = (10, 15)
MC = 144
bf16 = jnp.bfloat16


def _sigmoid(x):
    return 0.5 * (jnp.tanh(0.5 * x) + 1.0)


def _full(shape):
    nd = len(shape)
    return pl.BlockSpec(tuple(shape), lambda _n=nd: (0,) * _n)


def _mm576(lhs_ref, col0, mxu, lsr):
    for j, mc in enumerate(range(0, T * 16, MC)):
        chunk = lhs_ref[pl.ds(mc, MC), pl.ds(col0, 256)].astype(bf16)
        pltpu.matmul_acc_lhs(mc // 4, chunk, mxu,
                             load_staged_rhs=lsr if j == 0 else None)


def _pop576(out_ref, col0, mxu, bias):
    for mc in range(0, T * 16, MC):
        v = pltpu.matmul_pop(mc // 4, (MC, 256), jnp.float32, mxu)
        out_ref[pl.ds(mc, MC), pl.ds(col0, 256)] = v + bias


def _cell(g0, g1, g2, g3, c_prev):
    i = _sigmoid(g0)
    f = _sigmoid(g1)
    g = jnp.tanh(g2)
    o = _sigmoid(g3)
    c = f * c_prev + i * g
    return o * jnp.tanh(c), c


def _lstm_bidir_loop(xp_scr, whh16_scr, store_fwd, store_bwd, Bp, H):
    f32 = jnp.float32
    z = jnp.zeros((Bp, H), f32)
    z16 = jnp.zeros((Bp, H), bf16)

    def push(dirn, tile, msr, mxu):
        pltpu.matmul_push_rhs(
            whh16_scr[dirn, :, pl.ds((2 * mxu + tile) * 256, 256)], msr, mxu)

    for mxu in range(2):
        push(0, 0, 0, mxu)
        pltpu.matmul_acc_lhs(0, z16, mxu, load_staged_rhs=0)

    def gates_f(rf):
        xpf = xp_scr[pl.ds(rf, Bp), :]
        return (pltpu.matmul_pop(0, (Bp, 256), f32, 0) + xpf[:, 0:256],
                pltpu.matmul_pop(8, (Bp, 256), f32, 0) + xpf[:, 256:512],
                pltpu.matmul_pop(0, (Bp, 256), f32, 1) + xpf[:, 512:768],
                pltpu.matmul_pop(8, (Bp, 256), f32, 1) + xpf[:, 768:1024])

    def gates_b(rb):
        xpb = xp_scr[pl.ds(rb, Bp), :]
        return (pltpu.matmul_pop(16, (Bp, 256), f32, 0) + xpb[:, 1024:1280],
                pltpu.matmul_pop(24, (Bp, 256), f32, 0) + xpb[:, 1280:1536],
                pltpu.matmul_pop(16, (Bp, 256), f32, 1) + xpb[:, 1536:1792],
                pltpu.matmul_pop(24, (Bp, 256), f32, 1) + xpb[:, 1792:2048])

    def step_even(s0, hf, cf, hb, cb):
        rf = pl.multiple_of(s0 * Bp, Bp)
        rb = pl.multiple_of((T - 1 - s0) * Bp, Bp)
        hf16 = hf.astype(bf16)
        hb16 = hb.astype(bf16)
        for mxu in range(2):
            pltpu.matmul_acc_lhs(0, hf16, mxu, load_staged_rhs=None)
            push(0, 1, 1, mxu)
            pltpu.matmul_acc_lhs(8, hf16, mxu, load_staged_rhs=1)
            push(1, 0, 0, mxu)
            pltpu.matmul_acc_lhs(16, hb16, mxu, load_staged_rhs=0)
            push(1, 1, 1, mxu)
            pltpu.matmul_acc_lhs(24, hb16, mxu, load_staged_rhs=1)
        hf, cf = _cell(*gates_f(rf), cf)
        store_fwd(rf, s0, hf)
        hb, cb = _cell(*gates_b(rb), cb)
        store_bwd(rb, T - 1 - s0, hb)
        return hf, cf, hb, cb

    def step_odd(s1, hf, cf, hb, cb):
        rf = pl.multiple_of(s1 * Bp, Bp)
        rb = pl.multiple_of((T - 1 - s1) * Bp, Bp)
        hf16 = hf.astype(bf16)
        hb16 = hb.astype(bf16)
        for mxu in range(2):
            pltpu.matmul_acc_lhs(24, hb16, mxu, load_staged_rhs=None)
            push(1, 0, 0, mxu)
            pltpu.matmul_acc_lhs(16, hb16, mxu, load_staged_rhs=0)
            push(0, 1, 1, mxu)
            pltpu.matmul_acc_lhs(8, hf16, mxu, load_staged_rhs=1)
            push(0, 0, 0, mxu)
            pltpu.matmul_acc_lhs(0, hf16, mxu, load_staged_rhs=0)
        hb, cb = _cell(*gates_b(rb), cb)
        store_bwd(rb, T - 1 - s1, hb)
        hf, cf = _cell(*gates_f(rf), cf)
        store_fwd(rf, s1, hf)
        return hf, cf, hb, cb

    def body(p4, carry):
        s0 = 4 * p4
        carry = step_even(s0, *carry)
        carry = step_odd(s0 + 1, *carry)
        carry = step_even(s0 + 2, *carry)
        carry = step_odd(s0 + 3, *carry)
        return carry

    lax.fori_loop(0, T // 4, body, (z, z, z, z))


def _fused_kernel(sp_ref, cw_ref, cb_ref,
                  wih1_ref, b1_ref, whh1f_ref, whh1b_ref,
                  wih2f_ref, wih2b_ref, b2_ref, whh2f_ref, whh2b_ref,
                  fc1wf_hbm, fc1wb_hbm, fc1b_ref,
                  fc2w_ref, fc2b_ref, fc3w_ref, fc3b_ref,
                  o_ref,
                  pscr, cwscr, feat_scr, xp_scr, whh16_scr,
                  l1_scr, h2f_scr, h2b_scr, fc1wf_scr, fc1wb_scr,
                  semf, semb, *, Bp, H, FCH):
    f32 = jnp.float32
    CK2 = sp_ref.shape[1]
    K2 = CK2 // 4

    pltpu.make_async_copy(fc1wf_hbm, fc1wf_scr, semf).start()
    pltpu.make_async_copy(fc1wb_hbm, fc1wb_scr, semb).start()

    pscr[...] = jnp.zeros((T * Bp, 256), bf16)
    pscr[:, pl.ds(0, CK2)] = sp_ref[...]

    for p in range(POOL):
        cwscr[...] = jnp.zeros((256, 256), f32)
        r0 = 0
        for bi, K in enumerate(CONV_KS):
            off = p + (CONV_KS[-1] - 1) // 2 - (K - 1) // 2
            for c in range(4):
                rows = pl.ds(c * K2 + off, K)
                src = cw_ref[pl.ds(r0 + c * K, K), :]
                if bi == 0:
                    cwscr[rows, :] = src
                else:
                    cwscr[rows, :] = cwscr[rows, :] + src
            r0 += 4 * K
        mxu = p % 2
        pltpu.matmul_push_rhs(cwscr[...].astype(bf16), 0, mxu)
        _mm576(pscr, 0, mxu, 0)
        for mc in range(0, T * Bp, MC):
            v = pltpu.matmul_pop(mc // 4, (MC, 256), f32, mxu)
            if p == 0:
                feat_scr[pl.ds(mc, MC), :] = v
            elif p == 1:
                feat_scr[pl.ds(mc, MC), :] = jnp.maximum(
                    feat_scr[pl.ds(mc, MC), :], v)
            else:
                feat_scr[pl.ds(mc, MC), :] = jnp.maximum(
                    jnp.maximum(feat_scr[pl.ds(mc, MC), :], v) + cb_ref[...],
                    0.0)

    for n in range(8):
        mxu = n % 2
        pltpu.matmul_push_rhs(
            wih1_ref[:, pl.ds(n * 256, 256)].astype(bf16), 0, mxu)
        _mm576(feat_scr, 0, mxu, 0)
        _pop576(xp_scr, n * 256, mxu, b1_ref[0, pl.ds(n * 256, 256)][None, :])

    whh16_scr[0] = whh1f_ref[...].astype(bf16)
    whh16_scr[1] = whh1b_ref[...].astype(bf16)

    def store_fwd1(r, t, h):
        l1_scr[0, pl.ds(r, Bp), :] = h

    def store_bwd1(r, t, h):
        l1_scr[1, pl.ds(r, Bp), :] = h

    _lstm_bidir_loop(xp_scr, whh16_scr, store_fwd1, store_bwd1, Bp, H)

    for n in range(8):
        mxu = n % 2
        pltpu.matmul_push_rhs(
            wih2f_ref[:, pl.ds(n * 256, 256)].astype(bf16), 0, mxu)
        pltpu.matmul_push_rhs(
            wih2b_ref[:, pl.ds(n * 256, 256)].astype(bf16), 1, mxu)
        _mm576(l1_scr.at[0], 0, mxu, 0)
        _mm576(l1_scr.at[1], 0, mxu, 1)
        _pop576(xp_scr, n * 256, mxu, b2_ref[0, pl.ds(n * 256, 256)][None, :])

    whh16_scr[0] = whh2f_ref[...].astype(bf16)
    whh16_scr[1] = whh2b_ref[...].astype(bf16)

    def store_fwd(r, t, h):
        h2f_scr[:, pl.ds(pl.multiple_of(t * H, H), H)] = h

    def store_bwd(r, t, h):
        h2b_scr[:, pl.ds(pl.multiple_of(t * H, H), H)] = h

    _lstm_bidir_loop(xp_scr, whh16_scr, store_fwd, store_bwd, Bp, H)

    pltpu.make_async_copy(fc1wf_hbm, fc1wf_scr, semf).wait()
    pltpu.make_async_copy(fc1wb_hbm, fc1wb_scr, semb).wait()
    for kt in range(T):
        msr = kt % 2
        pltpu.matmul_push_rhs(
            fc1wf_scr[pl.ds(kt * 256, 256), :].astype(bf16), msr, 0)
        pltpu.matmul_acc_lhs(0, h2f_scr[:, pl.ds(kt * 256, 256)].astype(bf16),
                             0, load_staged_rhs=msr)
        pltpu.matmul_push_rhs(
            fc1wb_scr[pl.ds(kt * 256, 256), :].astype(bf16), msr, 1)
        pltpu.matmul_acc_lhs(0, h2b_scr[:, pl.ds(kt * 256, 256)].astype(bf16),
                             1, load_staged_rhs=msr)
    acc = (pltpu.matmul_pop(0, (Bp, FCH), f32, 0)
           + pltpu.matmul_pop(0, (Bp, FCH), f32, 1))

    y = jnp.maximum(acc + fc1b_ref[...], 0.0)
    pltpu.matmul_push_rhs(fc2w_ref[...].astype(bf16), 0, 0)
    pltpu.matmul_acc_lhs(0, y.astype(bf16), 0, load_staged_rhs=0)
    y = jnp.maximum(pltpu.matmul_pop(0, (Bp, FCH), f32, 0)
                    + fc2b_ref[...], 0.0)
    o_ref[...] = jnp.sum(y * fc3w_ref[...], axis=1, keepdims=True) + fc3b_ref[...]


def kernel(x, cw, cb, wih1, b1, whh1f, whh1b, wih2f, wih2b, b2, whh2f, whh2b,
           fc1wf, fc1wb, fc1b, fc2w, fc2b, fc3w, fc3b):
    f32 = jnp.float32
    B, L, Cin = x.shape
    H = whh1f.shape[0]
    FCH = fc2w.shape[0]
    C = cw.shape[1]
    Bp = max(8, (B + 7) // 8 * 8)

    xb = jnp.pad(x.astype(f32), ((0, Bp - B), (0, 0), (0, 0)))
    x_bcl = jnp.transpose(xb, (0, 2, 1))

    Kmax = max(CONV_KS)
    K2 = Kmax + POOL - 1
    pad_big = (Kmax - 1) // 2
    xpd = jnp.pad(x_bcl.astype(bf16),
                  ((0, 0), (0, 0), (pad_big, Kmax - 1 - pad_big)))
    idx = POOL * jnp.arange(T)[:, None] + jnp.arange(K2)[None, :]
    pt = xpd[:, :, idx]
    spatch = jnp.transpose(pt, (2, 0, 1, 3)).reshape(T * Bp, Cin * K2)
    CK2 = Cin * K2

    out = pl.pallas_call(
        functools.partial(_fused_kernel, Bp=Bp, H=H, FCH=FCH),
        out_shape=jax.ShapeDtypeStruct((Bp, 1), f32),
        in_specs=[
            _full((T * Bp, CK2)),
            _full((cw.shape[0], C)), _full((1, C)),
            _full((C, 8 * H)), _full((1, 8 * H)),
            _full((H, 4 * H)), _full((H, 4 * H)),
            _full((H, 8 * H)), _full((H, 8 * H)), _full((1, 8 * H)),
            _full((H, 4 * H)), _full((H, 4 * H)),
            pl.BlockSpec(memory_space=pl.ANY),
            pl.BlockSpec(memory_space=pl.ANY),
            _full((1, FCH)),
            _full((FCH, FCH)), _full((1, FCH)),
            _full((1, FCH)), _full((1, 1)),
        ],
        out_specs=_full((Bp, 1)),
        scratch_shapes=[
            pltpu.VMEM((T * Bp, 256), bf16),
            pltpu.VMEM((256, 256), f32),
            pltpu.VMEM((T * Bp, C), f32),
            pltpu.VMEM((T * Bp, 8 * H), f32),
            pltpu.VMEM((2, H, 4 * H), bf16),
            pltpu.VMEM((2, T * Bp, H), f32),
            pltpu.VMEM((Bp, T * H), f32),
            pltpu.VMEM((Bp, T * H), f32),
            pltpu.VMEM((T * H, FCH), f32),
            pltpu.VMEM((T * H, FCH), f32),
            pltpu.SemaphoreType.DMA,
            pltpu.SemaphoreType.DMA,
        ],
        grid=(),
    )(spatch, cw, cb, wih1, b1, whh1f, whh1b,
      wih2f, wih2b, b2, whh2f, whh2b, fc1wf, fc1wb, fc1b,
      fc2w, fc2b, fc3w, fc3b)

    return out[:B, 0]


# speedup vs baseline: 1.1116x; 1.0862x over previous
import functools

import jax
import jax.numpy as jnp
from jax import lax
from jax.experimental import pallas as pl
from jax.experimental.pallas import tpu as pltpu

T = 36
POOL = 3
CONV_KS = (10, 15)
MC = 144
bf16 = jnp.bfloat16


def _sigmoid(x):
    return 0.5 * (jnp.tanh(0.5 * x) + 1.0)


def _full(shape):
    nd = len(shape)
    return pl.BlockSpec(tuple(shape), lambda _n=nd: (0,) * _n)


def _mm576(lhs_ref, col0, mxu, lsr):
    for j, mc in enumerate(range(0, T * 16, MC)):
        chunk = lhs_ref[pl.ds(mc, MC), pl.ds(col0, 256)].astype(bf16)
        pltpu.matmul_acc_lhs(mc // 4, chunk, mxu,
                             load_staged_rhs=lsr if j == 0 else None)


def _pop576(out_ref, col0, mxu, bias):
    for mc in range(0, T * 16, MC):
        v = pltpu.matmul_pop(mc // 4, (MC, 256), jnp.float32, mxu)
        out_ref[pl.ds(mc, MC), pl.ds(col0, 256)] = v + bias


def _cell(g0, g1, g2, g3, c_prev):
    i = _sigmoid(g0)
    f = _sigmoid(g1)
    g = jnp.tanh(g2)
    o = _sigmoid(g3)
    c = f * c_prev + i * g
    return o * jnp.tanh(c), c


def _lstm_bidir_loop(xp_scr, whh16_scr, store_fwd, store_bwd, Bp, H):
    f32 = jnp.float32
    z = jnp.zeros((Bp, H), f32)
    z16 = jnp.zeros((Bp, H), bf16)

    def push(dirn, tile, msr, mxu):
        pltpu.matmul_push_rhs(
            whh16_scr[dirn, :, pl.ds((2 * mxu + tile) * 256, 256)], msr, mxu)

    for mxu in range(2):
        push(0, 0, 0, mxu)
        pltpu.matmul_acc_lhs(0, z16, mxu, load_staged_rhs=0)

    def gates_f(rf):
        xpf = xp_scr[pl.ds(rf, Bp), :]
        return (pltpu.matmul_pop(0, (Bp, 256), f32, 0) + xpf[:, 0:256],
                pltpu.matmul_pop(8, (Bp, 256), f32, 0) + xpf[:, 256:512],
                pltpu.matmul_pop(0, (Bp, 256), f32, 1) + xpf[:, 512:768],
                pltpu.matmul_pop(8, (Bp, 256), f32, 1) + xpf[:, 768:1024])

    def gates_b(rb):
        xpb = xp_scr[pl.ds(rb, Bp), :]
        return (pltpu.matmul_pop(16, (Bp, 256), f32, 0) + xpb[:, 1024:1280],
                pltpu.matmul_pop(24, (Bp, 256), f32, 0) + xpb[:, 1280:1536],
                pltpu.matmul_pop(16, (Bp, 256), f32, 1) + xpb[:, 1536:1792],
                pltpu.matmul_pop(24, (Bp, 256), f32, 1) + xpb[:, 1792:2048])

    def step_even(s0, hf, cf, hb, cb):
        rf = pl.multiple_of(s0 * Bp, Bp)
        rb = pl.multiple_of((T - 1 - s0) * Bp, Bp)
        hf16 = hf.astype(bf16)
        hb16 = hb.astype(bf16)
        for mxu in range(2):
            pltpu.matmul_acc_lhs(0, hf16, mxu, load_staged_rhs=None)
            push(0, 1, 1, mxu)
            pltpu.matmul_acc_lhs(8, hf16, mxu, load_staged_rhs=1)
            push(1, 0, 0, mxu)
            pltpu.matmul_acc_lhs(16, hb16, mxu, load_staged_rhs=0)
            push(1, 1, 1, mxu)
            pltpu.matmul_acc_lhs(24, hb16, mxu, load_staged_rhs=1)
        hf, cf = _cell(*gates_f(rf), cf)
        store_fwd(rf, s0, hf)
        hb, cb = _cell(*gates_b(rb), cb)
        store_bwd(rb, T - 1 - s0, hb)
        return hf, cf, hb, cb

    def step_odd(s1, hf, cf, hb, cb):
        rf = pl.multiple_of(s1 * Bp, Bp)
        rb = pl.multiple_of((T - 1 - s1) * Bp, Bp)
        hf16 = hf.astype(bf16)
        hb16 = hb.astype(bf16)
        for mxu in range(2):
            pltpu.matmul_acc_lhs(24, hb16, mxu, load_staged_rhs=None)
            push(1, 0, 0, mxu)
            pltpu.matmul_acc_lhs(16, hb16, mxu, load_staged_rhs=0)
            push(0, 1, 1, mxu)
            pltpu.matmul_acc_lhs(8, hf16, mxu, load_staged_rhs=1)
            push(0, 0, 0, mxu)
            pltpu.matmul_acc_lhs(0, hf16, mxu, load_staged_rhs=0)
        hb, cb = _cell(*gates_b(rb), cb)
        store_bwd(rb, T - 1 - s1, hb)
        hf, cf = _cell(*gates_f(rf), cf)
        store_fwd(rf, s1, hf)
        return hf, cf, hb, cb

    def body(p4, carry):
        s0 = 4 * p4
        carry = step_even(s0, *carry)
        carry = step_odd(s0 + 1, *carry)
        carry = step_even(s0 + 2, *carry)
        carry = step_odd(s0 + 3, *carry)
        return carry

    lax.fori_loop(0, T // 4, body, (z, z, z, z))


def _fused_kernel(sp_ref, cw_ref, cb_ref,
                  wih1_ref, b1_ref, whh1f_ref, whh1b_ref,
                  wih2f_ref, wih2b_ref, b2_ref, whh2f_ref, whh2b_ref,
                  fc1wf_hbm, fc1wb_hbm, fc1b_ref,
                  fc2w_ref, fc2b_ref, fc3w_ref, fc3b_ref,
                  o_ref,
                  pscr, cwscr, feat_scr, xp_scr, whh16_scr,
                  l1_scr, h2f_scr, h2b_scr, fc1wf_scr, fc1wb_scr,
                  semf, semb, *, Bp, H, FCH):
    f32 = jnp.float32
    CK2 = sp_ref.shape[1]
    K2 = CK2 // 4

    pltpu.make_async_copy(fc1wf_hbm, fc1wf_scr, semf).start()
    pltpu.make_async_copy(fc1wb_hbm, fc1wb_scr, semb).start()

    pscr[...] = jnp.zeros((T * Bp, 256), bf16)
    pscr[:, pl.ds(0, CK2)] = sp_ref[...]

    for p in range(POOL):
        cwscr[...] = jnp.zeros((256, 256), f32)
        r0 = 0
        for bi, K in enumerate(CONV_KS):
            off = p + (CONV_KS[-1] - 1) // 2 - (K - 1) // 2
            for c in range(4):
                rows = pl.ds(c * K2 + off, K)
                src = cw_ref[pl.ds(r0 + c * K, K), :]
                if bi == 0:
                    cwscr[rows, :] = src
                else:
                    cwscr[rows, :] = cwscr[rows, :] + src
            r0 += 4 * K
        mxu = p % 2
        pltpu.matmul_push_rhs(cwscr[...].astype(bf16), 0, mxu)
        _mm576(pscr, 0, mxu, 0)
        for mc in range(0, T * Bp, MC):
            v = pltpu.matmul_pop(mc // 4, (MC, 256), f32, mxu)
            if p == 0:
                feat_scr[pl.ds(mc, MC), :] = v
            elif p == 1:
                feat_scr[pl.ds(mc, MC), :] = jnp.maximum(
                    feat_scr[pl.ds(mc, MC), :], v)
            else:
                feat_scr[pl.ds(mc, MC), :] = jnp.maximum(
                    jnp.maximum(feat_scr[pl.ds(mc, MC), :], v) + cb_ref[...],
                    0.0)

    for n in range(8):
        mxu = n % 2
        pltpu.matmul_push_rhs(
            wih1_ref[:, pl.ds(n * 256, 256)].astype(bf16), 0, mxu)
        _mm576(feat_scr, 0, mxu, 0)
        _pop576(xp_scr, n * 256, mxu, b1_ref[0, pl.ds(n * 256, 256)][None, :])

    whh16_scr[0] = whh1f_ref[...].astype(bf16)
    whh16_scr[1] = whh1b_ref[...].astype(bf16)

    def store_fwd1(r, t, h):
        l1_scr[0, pl.ds(r, Bp), :] = h

    def store_bwd1(r, t, h):
        l1_scr[1, pl.ds(r, Bp), :] = h

    _lstm_bidir_loop(xp_scr, whh16_scr, store_fwd1, store_bwd1, Bp, H)

    for n in range(8):
        mxu = n % 2
        pltpu.matmul_push_rhs(
            wih2f_ref[:, pl.ds(n * 256, 256)].astype(bf16), 0, mxu)
        pltpu.matmul_push_rhs(
            wih2b_ref[:, pl.ds(n * 256, 256)].astype(bf16), 1, mxu)
        _mm576(l1_scr.at[0], 0, mxu, 0)
        _mm576(l1_scr.at[1], 0, mxu, 1)
        _pop576(xp_scr, n * 256, mxu, b2_ref[0, pl.ds(n * 256, 256)][None, :])

    whh16_scr[0] = whh2f_ref[...].astype(bf16)
    whh16_scr[1] = whh2b_ref[...].astype(bf16)

    def store_fwd(r, t, h):
        h2f_scr[:, pl.ds(pl.multiple_of(t * H, H), H)] = h

    def store_bwd(r, t, h):
        h2b_scr[:, pl.ds(pl.multiple_of(t * H, H), H)] = h

    _lstm_bidir_loop(xp_scr, whh16_scr, store_fwd, store_bwd, Bp, H)

    pltpu.make_async_copy(fc1wf_hbm, fc1wf_scr, semf).wait()
    pltpu.make_async_copy(fc1wb_hbm, fc1wb_scr, semb).wait()
    for kt in range(T):
        msr = kt % 2
        pltpu.matmul_push_rhs(
            fc1wf_scr[pl.ds(kt * 256, 256), :].astype(bf16), msr, 0)
        pltpu.matmul_acc_lhs(0, h2f_scr[:, pl.ds(kt * 256, 256)].astype(bf16),
                             0, load_staged_rhs=msr)
        pltpu.matmul_push_rhs(
            fc1wb_scr[pl.ds(kt * 256, 256), :].astype(bf16), msr, 1)
        pltpu.matmul_acc_lhs(0, h2b_scr[:, pl.ds(kt * 256, 256)].astype(bf16),
                             1, load_staged_rhs=msr)
    acc = (pltpu.matmul_pop(0, (Bp, FCH), f32, 0)
           + pltpu.matmul_pop(0, (Bp, FCH), f32, 1))

    y = jnp.maximum(acc + fc1b_ref[...], 0.0)
    pltpu.matmul_push_rhs(fc2w_ref[...].astype(bf16), 0, 0)
    pltpu.matmul_acc_lhs(0, y.astype(bf16), 0, load_staged_rhs=0)
    y = jnp.maximum(pltpu.matmul_pop(0, (Bp, FCH), f32, 0)
                    + fc2b_ref[...], 0.0)
    o_ref[...] = jnp.sum(y * fc3w_ref[...], axis=1, keepdims=True) + fc3b_ref[...]


def kernel(x, cw, cb, wih1, b1, whh1f, whh1b, wih2f, wih2b, b2, whh2f, whh2b,
           fc1wf, fc1wb, fc1b, fc2w, fc2b, fc3w, fc3b):
    f32 = jnp.float32
    B, L, Cin = x.shape
    H = whh1f.shape[0]
    FCH = fc2w.shape[0]
    C = cw.shape[1]
    Bp = max(8, (B + 7) // 8 * 8)

    xb = jnp.pad(x.astype(f32), ((0, Bp - B), (0, 0), (0, 0)))
    x_bcl = jnp.transpose(xb, (0, 2, 1))

    Kmax = max(CONV_KS)
    K2 = Kmax + POOL
    pad_big = (Kmax - 1) // 2
    NW = K2 // POOL
    Lp = POOL * (T + NW - 1)
    xpd = jnp.pad(x_bcl.astype(bf16),
                  ((0, 0), (0, 0), (pad_big, Lp - L - pad_big)))
    xt3 = jnp.transpose(xpd, (2, 0, 1)).reshape(Lp // POOL, POOL * Bp * Cin)
    slabs = jnp.concatenate([xt3[w:w + T] for w in range(NW)], axis=1)
    spatch = jnp.transpose(slabs.reshape(T, NW, POOL, Bp, Cin),
                           (0, 3, 4, 1, 2)).reshape(T * Bp, Cin * K2)
    CK2 = Cin * K2

    out = pl.pallas_call(
        functools.partial(_fused_kernel, Bp=Bp, H=H, FCH=FCH),
        out_shape=jax.ShapeDtypeStruct((Bp, 1), f32),
        in_specs=[
            _full((T * Bp, CK2)),
            _full((cw.shape[0], C)), _full((1, C)),
            _full((C, 8 * H)), _full((1, 8 * H)),
            _full((H, 4 * H)), _full((H, 4 * H)),
            _full((H, 8 * H)), _full((H, 8 * H)), _full((1, 8 * H)),
            _full((H, 4 * H)), _full((H, 4 * H)),
            pl.BlockSpec(memory_space=pl.ANY),
            pl.BlockSpec(memory_space=pl.ANY),
            _full((1, FCH)),
            _full((FCH, FCH)), _full((1, FCH)),
            _full((1, FCH)), _full((1, 1)),
        ],
        out_specs=_full((Bp, 1)),
        scratch_shapes=[
            pltpu.VMEM((T * Bp, 256), bf16),
            pltpu.VMEM((256, 256), f32),
            pltpu.VMEM((T * Bp, C), f32),
            pltpu.VMEM((T * Bp, 8 * H), f32),
            pltpu.VMEM((2, H, 4 * H), bf16),
            pltpu.VMEM((2, T * Bp, H), f32),
            pltpu.VMEM((Bp, T * H), f32),
            pltpu.VMEM((Bp, T * H), f32),
            pltpu.VMEM((T * H, FCH), f32),
            pltpu.VMEM((T * H, FCH), f32),
            pltpu.SemaphoreType.DMA,
            pltpu.SemaphoreType.DMA,
        ],
        grid=(),
    )(spatch, cw, cb, wih1, b1, whh1f, whh1b,
      wih2f, wih2b, b2, whh2f, whh2b, fc1wf, fc1wb, fc1b,
      fc2w, fc2b, fc3w, fc3b)

    return out[:B, 0]


# speedup vs baseline: 1.1167x; 1.0046x over previous
import functools

import jax
import jax.numpy as jnp
from jax import lax
from jax.experimental import pallas as pl
from jax.experimental.pallas import tpu as pltpu

T = 36
POOL = 3
CONV_KS = (10, 15)
MC = 144
bf16 = jnp.bfloat16


def _sigmoid(x):
    return 0.5 * (jnp.tanh(0.5 * x) + 1.0)


def _full(shape):
    nd = len(shape)
    return pl.BlockSpec(tuple(shape), lambda _n=nd: (0,) * _n)


def _mm576(lhs_ref, col0, mxu, lsr):
    for j, mc in enumerate(range(0, T * 16, MC)):
        chunk = lhs_ref[pl.ds(mc, MC), pl.ds(col0, 256)].astype(bf16)
        pltpu.matmul_acc_lhs(mc // 4, chunk, mxu,
                             load_staged_rhs=lsr if j == 0 else None)


def _pop576(out_ref, col0, mxu, bias):
    for mc in range(0, T * 16, MC):
        v = pltpu.matmul_pop(mc // 4, (MC, 256), jnp.float32, mxu)
        out_ref[pl.ds(mc, MC), pl.ds(col0, 256)] = v + bias


def _cell(g0, g1, g2, g3, c_prev):
    i = _sigmoid(g0)
    f = _sigmoid(g1)
    g = jnp.tanh(g2)
    o = _sigmoid(g3)
    c = f * c_prev + i * g
    return o * jnp.tanh(c), c


def _lstm_bidir_loop(xp_scr, whh16_scr, store_fwd, store_bwd, Bp, H):
    f32 = jnp.float32
    z = jnp.zeros((Bp, H), f32)
    z16 = jnp.zeros((Bp, H), bf16)

    def push(dirn, tile, msr, mxu):
        pltpu.matmul_push_rhs(
            whh16_scr[dirn, :, pl.ds((2 * mxu + tile) * 256, 256)], msr, mxu)

    for mxu in range(2):
        push(0, 0, 0, mxu)
        pltpu.matmul_acc_lhs(0, z16, mxu, load_staged_rhs=0)

    def gates_f(rf):
        xpf = xp_scr[pl.ds(rf, Bp), :]
        return (pltpu.matmul_pop(0, (Bp, 256), f32, 0) + xpf[:, 0:256],
                pltpu.matmul_pop(8, (Bp, 256), f32, 0) + xpf[:, 256:512],
                pltpu.matmul_pop(0, (Bp, 256), f32, 1) + xpf[:, 512:768],
                pltpu.matmul_pop(8, (Bp, 256), f32, 1) + xpf[:, 768:1024])

    def gates_b(rb):
        xpb = xp_scr[pl.ds(rb, Bp), :]
        return (pltpu.matmul_pop(16, (Bp, 256), f32, 0) + xpb[:, 1024:1280],
                pltpu.matmul_pop(24, (Bp, 256), f32, 0) + xpb[:, 1280:1536],
                pltpu.matmul_pop(16, (Bp, 256), f32, 1) + xpb[:, 1536:1792],
                pltpu.matmul_pop(24, (Bp, 256), f32, 1) + xpb[:, 1792:2048])

    def step_even(s0, hf, cf, hb, cb):
        rf = pl.multiple_of(s0 * Bp, Bp)
        rb = pl.multiple_of((T - 1 - s0) * Bp, Bp)
        hf16 = hf.astype(bf16)
        hb16 = hb.astype(bf16)
        for mxu in range(2):
            pltpu.matmul_acc_lhs(0, hf16, mxu, load_staged_rhs=None)
            push(0, 1, 1, mxu)
            pltpu.matmul_acc_lhs(8, hf16, mxu, load_staged_rhs=1)
            push(1, 0, 0, mxu)
            pltpu.matmul_acc_lhs(16, hb16, mxu, load_staged_rhs=0)
            push(1, 1, 1, mxu)
            pltpu.matmul_acc_lhs(24, hb16, mxu, load_staged_rhs=1)
        hf, cf = _cell(*gates_f(rf), cf)
        store_fwd(rf, s0, hf)
        hb, cb = _cell(*gates_b(rb), cb)
        store_bwd(rb, T - 1 - s0, hb)
        return hf, cf, hb, cb

    def step_odd(s1, hf, cf, hb, cb):
        rf = pl.multiple_of(s1 * Bp, Bp)
        rb = pl.multiple_of((T - 1 - s1) * Bp, Bp)
        hf16 = hf.astype(bf16)
        hb16 = hb.astype(bf16)
        for mxu in range(2):
            pltpu.matmul_acc_lhs(24, hb16, mxu, load_staged_rhs=None)
            push(1, 0, 0, mxu)
            pltpu.matmul_acc_lhs(16, hb16, mxu, load_staged_rhs=0)
            push(0, 1, 1, mxu)
            pltpu.matmul_acc_lhs(8, hf16, mxu, load_staged_rhs=1)
            push(0, 0, 0, mxu)
            pltpu.matmul_acc_lhs(0, hf16, mxu, load_staged_rhs=0)
        hb, cb = _cell(*gates_b(rb), cb)
        store_bwd(rb, T - 1 - s1, hb)
        hf, cf = _cell(*gates_f(rf), cf)
        store_fwd(rf, s1, hf)
        return hf, cf, hb, cb

    def body(p4, carry):
        s0 = 4 * p4
        carry = step_even(s0, *carry)
        carry = step_odd(s0 + 1, *carry)
        carry = step_even(s0 + 2, *carry)
        carry = step_odd(s0 + 3, *carry)
        return carry

    lax.fori_loop(0, T // 4, body, (z, z, z, z))


def _fused_kernel(sp_ref, cw_ref, cb_ref,
                  wih1_hbm, b1_ref, whh1f_hbm, whh1b_hbm,
                  wih2f_hbm, wih2b_hbm, b2_ref, whh2f_hbm, whh2b_hbm,
                  fc1wf_hbm, fc1wb_hbm, fc1b_ref,
                  fc2w_ref, fc2b_ref, fc3w_ref, fc3b_ref,
                  o_ref,
                  pscr, cwscr, feat_scr, xp_scr, whh16_scr,
                  l1_scr, h2f_scr, h2b_scr, fc1wf_scr, fc1wb_scr,
                  wih1_ref, whh1f_ref, whh1b_ref, wih2f_ref, wih2b_ref,
                  whh2f_ref, whh2b_ref,
                  semf, semb, sem1, semh1, sem2, semh2, *, Bp, H, FCH):
    f32 = jnp.float32
    CK2 = sp_ref.shape[1]
    K2 = CK2 // 4

    pltpu.make_async_copy(wih1_hbm, wih1_ref, sem1).start()
    pltpu.make_async_copy(whh1f_hbm, whh1f_ref, semh1).start()
    pltpu.make_async_copy(whh1b_hbm, whh1b_ref, semh1).start()
    pltpu.make_async_copy(wih2f_hbm, wih2f_ref, sem2).start()
    pltpu.make_async_copy(wih2b_hbm, wih2b_ref, sem2).start()
    pltpu.make_async_copy(whh2f_hbm, whh2f_ref, semh2).start()
    pltpu.make_async_copy(whh2b_hbm, whh2b_ref, semh2).start()
    pltpu.make_async_copy(fc1wf_hbm, fc1wf_scr, semf).start()
    pltpu.make_async_copy(fc1wb_hbm, fc1wb_scr, semb).start()

    pscr[...] = jnp.zeros((T * Bp, 256), bf16)
    pscr[:, pl.ds(0, CK2)] = sp_ref[...]

    for p in range(POOL):
        cwscr[...] = jnp.zeros((256, 256), f32)
        r0 = 0
        for bi, K in enumerate(CONV_KS):
            off = p + (CONV_KS[-1] - 1) // 2 - (K - 1) // 2
            for c in range(4):
                rows = pl.ds(c * K2 + off, K)
                src = cw_ref[pl.ds(r0 + c * K, K), :]
                if bi == 0:
                    cwscr[rows, :] = src
                else:
                    cwscr[rows, :] = cwscr[rows, :] + src
            r0 += 4 * K
        mxu = p % 2
        pltpu.matmul_push_rhs(cwscr[...].astype(bf16), 0, mxu)
        _mm576(pscr, 0, mxu, 0)
        for mc in range(0, T * Bp, MC):
            v = pltpu.matmul_pop(mc // 4, (MC, 256), f32, mxu)
            if p == 0:
                feat_scr[pl.ds(mc, MC), :] = v
            elif p == 1:
                feat_scr[pl.ds(mc, MC), :] = jnp.maximum(
                    feat_scr[pl.ds(mc, MC), :], v)
            else:
                feat_scr[pl.ds(mc, MC), :] = jnp.maximum(
                    jnp.maximum(feat_scr[pl.ds(mc, MC), :], v) + cb_ref[...],
                    0.0)

    pltpu.make_async_copy(wih1_hbm, wih1_ref, sem1).wait()
    for n in range(8):
        mxu = n % 2
        pltpu.matmul_push_rhs(
            wih1_ref[:, pl.ds(n * 256, 256)].astype(bf16), 0, mxu)
        _mm576(feat_scr, 0, mxu, 0)
        _pop576(xp_scr, n * 256, mxu, b1_ref[0, pl.ds(n * 256, 256)][None, :])

    pltpu.make_async_copy(whh1f_hbm, whh1f_ref, semh1).wait()
    pltpu.make_async_copy(whh1b_hbm, whh1b_ref, semh1).wait()
    whh16_scr[0] = whh1f_ref[...].astype(bf16)
    whh16_scr[1] = whh1b_ref[...].astype(bf16)

    def store_fwd1(r, t, h):
        l1_scr[0, pl.ds(r, Bp), :] = h

    def store_bwd1(r, t, h):
        l1_scr[1, pl.ds(r, Bp), :] = h

    _lstm_bidir_loop(xp_scr, whh16_scr, store_fwd1, store_bwd1, Bp, H)

    pltpu.make_async_copy(wih2f_hbm, wih2f_ref, sem2).wait()
    pltpu.make_async_copy(wih2b_hbm, wih2b_ref, sem2).wait()
    for n in range(8):
        mxu = n % 2
        pltpu.matmul_push_rhs(
            wih2f_ref[:, pl.ds(n * 256, 256)].astype(bf16), 0, mxu)
        pltpu.matmul_push_rhs(
            wih2b_ref[:, pl.ds(n * 256, 256)].astype(bf16), 1, mxu)
        _mm576(l1_scr.at[0], 0, mxu, 0)
        _mm576(l1_scr.at[1], 0, mxu, 1)
        _pop576(xp_scr, n * 256, mxu, b2_ref[0, pl.ds(n * 256, 256)][None, :])

    pltpu.make_async_copy(whh2f_hbm, whh2f_ref, semh2).wait()
    pltpu.make_async_copy(whh2b_hbm, whh2b_ref, semh2).wait()
    whh16_scr[0] = whh2f_ref[...].astype(bf16)
    whh16_scr[1] = whh2b_ref[...].astype(bf16)

    def store_fwd(r, t, h):
        h2f_scr[:, pl.ds(pl.multiple_of(t * H, H), H)] = h

    def store_bwd(r, t, h):
        h2b_scr[:, pl.ds(pl.multiple_of(t * H, H), H)] = h

    _lstm_bidir_loop(xp_scr, whh16_scr, store_fwd, store_bwd, Bp, H)

    pltpu.make_async_copy(fc1wf_hbm, fc1wf_scr, semf).wait()
    pltpu.make_async_copy(fc1wb_hbm, fc1wb_scr, semb).wait()
    for kt in range(T):
        msr = kt % 2
        pltpu.matmul_push_rhs(
            fc1wf_scr[pl.ds(kt * 256, 256), :].astype(bf16), msr, 0)
        pltpu.matmul_acc_lhs(0, h2f_scr[:, pl.ds(kt * 256, 256)].astype(bf16),
                             0, load_staged_rhs=msr)
        pltpu.matmul_push_rhs(
            fc1wb_scr[pl.ds(kt * 256, 256), :].astype(bf16), msr, 1)
        pltpu.matmul_acc_lhs(0, h2b_scr[:, pl.ds(kt * 256, 256)].astype(bf16),
                             1, load_staged_rhs=msr)
    acc = (pltpu.matmul_pop(0, (Bp, FCH), f32, 0)
           + pltpu.matmul_pop(0, (Bp, FCH), f32, 1))

    y = jnp.maximum(acc + fc1b_ref[...], 0.0)
    pltpu.matmul_push_rhs(fc2w_ref[...].astype(bf16), 0, 0)
    pltpu.matmul_acc_lhs(0, y.astype(bf16), 0, load_staged_rhs=0)
    y = jnp.maximum(pltpu.matmul_pop(0, (Bp, FCH), f32, 0)
                    + fc2b_ref[...], 0.0)
    o_ref[...] = jnp.sum(y * fc3w_ref[...], axis=1, keepdims=True) + fc3b_ref[...]


def kernel(x, cw, cb, wih1, b1, whh1f, whh1b, wih2f, wih2b, b2, whh2f, whh2b,
           fc1wf, fc1wb, fc1b, fc2w, fc2b, fc3w, fc3b):
    f32 = jnp.float32
    B, L, Cin = x.shape
    H = whh1f.shape[0]
    FCH = fc2w.shape[0]
    C = cw.shape[1]
    Bp = max(8, (B + 7) // 8 * 8)

    xb = jnp.pad(x.astype(f32), ((0, Bp - B), (0, 0), (0, 0)))
    x_bcl = jnp.transpose(xb, (0, 2, 1))

    Kmax = max(CONV_KS)
    K2 = Kmax + POOL
    pad_big = (Kmax - 1) // 2
    NW = K2 // POOL
    Lp = POOL * (T + NW - 1)
    xpd = jnp.pad(x_bcl.astype(bf16),
                  ((0, 0), (0, 0), (pad_big, Lp - L - pad_big)))
    xt3 = jnp.transpose(xpd, (2, 0, 1)).reshape(Lp // POOL, POOL * Bp * Cin)
    slabs = jnp.concatenate([xt3[w:w + T] for w in range(NW)], axis=1)
    spatch = jnp.transpose(slabs.reshape(T, NW, POOL, Bp, Cin),
                           (0, 3, 4, 1, 2)).reshape(T * Bp, Cin * K2)
    CK2 = Cin * K2

    out = pl.pallas_call(
        functools.partial(_fused_kernel, Bp=Bp, H=H, FCH=FCH),
        out_shape=jax.ShapeDtypeStruct((Bp, 1), f32),
        in_specs=[
            _full((T * Bp, CK2)),
            _full((cw.shape[0], C)), _full((1, C)),
            pl.BlockSpec(memory_space=pl.ANY), _full((1, 8 * H)),
            pl.BlockSpec(memory_space=pl.ANY),
            pl.BlockSpec(memory_space=pl.ANY),
            pl.BlockSpec(memory_space=pl.ANY),
            pl.BlockSpec(memory_space=pl.ANY), _full((1, 8 * H)),
            pl.BlockSpec(memory_space=pl.ANY),
            pl.BlockSpec(memory_space=pl.ANY),
            pl.BlockSpec(memory_space=pl.ANY),
            pl.BlockSpec(memory_space=pl.ANY),
            _full((1, FCH)),
            _full((FCH, FCH)), _full((1, FCH)),
            _full((1, FCH)), _full((1, 1)),
        ],
        out_specs=_full((Bp, 1)),
        scratch_shapes=[
            pltpu.VMEM((T * Bp, 256), bf16),
            pltpu.VMEM((256, 256), f32),
            pltpu.VMEM((T * Bp, C), f32),
            pltpu.VMEM((T * Bp, 8 * H), f32),
            pltpu.VMEM((2, H, 4 * H), bf16),
            pltpu.VMEM((2, T * Bp, H), f32),
            pltpu.VMEM((Bp, T * H), f32),
            pltpu.VMEM((Bp, T * H), f32),
            pltpu.VMEM((T * H, FCH), f32),
            pltpu.VMEM((T * H, FCH), f32),
            pltpu.VMEM((C, 8 * H), f32),
            pltpu.VMEM((H, 4 * H), f32),
            pltpu.VMEM((H, 4 * H), f32),
            pltpu.VMEM((H, 8 * H), f32),
            pltpu.VMEM((H, 8 * H), f32),
            pltpu.VMEM((H, 4 * H), f32),
            pltpu.VMEM((H, 4 * H), f32),
            pltpu.SemaphoreType.DMA,
            pltpu.SemaphoreType.DMA,
            pltpu.SemaphoreType.DMA,
            pltpu.SemaphoreType.DMA,
            pltpu.SemaphoreType.DMA,
            pltpu.SemaphoreType.DMA,
        ],
        grid=(),
    )(spatch, cw, cb, wih1, b1, whh1f, whh1b,
      wih2f, wih2b, b2, whh2f, whh2b, fc1wf, fc1wb, fc1b,
      fc2w, fc2b, fc3w, fc3b)

    return out[:B, 0]


# speedup vs baseline: 1.1271x; 1.0093x over previous
import functools

import jax
import jax.numpy as jnp
from jax import lax
from jax.experimental import pallas as pl
from jax.experimental.pallas import tpu as pltpu

T = 36
POOL = 3
CONV_KS = (10, 15)
MC = 144
bf16 = jnp.bfloat16


def _sigmoid(x):
    return 0.5 * (jnp.tanh(0.5 * x) + 1.0)


def _full(shape):
    nd = len(shape)
    return pl.BlockSpec(tuple(shape), lambda _n=nd: (0,) * _n)


def _mm576(lhs_ref, col0, mxu, lsr):
    for j, mc in enumerate(range(0, T * 16, MC)):
        chunk = lhs_ref[pl.ds(mc, MC), pl.ds(col0, 256)].astype(bf16)
        pltpu.matmul_acc_lhs(mc // 4, chunk, mxu,
                             load_staged_rhs=lsr if j == 0 else None)


def _pop576(out_ref, col0, mxu, bias):
    for mc in range(0, T * 16, MC):
        v = pltpu.matmul_pop(mc // 4, (MC, 256), jnp.float32, mxu)
        out_ref[pl.ds(mc, MC), pl.ds(col0, 256)] = v + bias


def _cell(g0, g1, g2, g3, c_prev):
    i = _sigmoid(g0)
    f = _sigmoid(g1)
    g = jnp.tanh(g2)
    o = _sigmoid(g3)
    c = f * c_prev + i * g
    return o * jnp.tanh(c), c


def _lstm_bidir_loop(xp_scr, whh16_scr, store_fwd, store_bwd, Bp, H):
    f32 = jnp.float32
    z = jnp.zeros((Bp, H), f32)
    z16 = jnp.zeros((Bp, H), bf16)

    def push(dirn, tile, msr, mxu):
        pltpu.matmul_push_rhs(
            whh16_scr[dirn, :, pl.ds((2 * mxu + tile) * 256, 256)], msr, mxu)

    for mxu in range(2):
        push(0, 0, 0, mxu)
        pltpu.matmul_acc_lhs(0, z16, mxu, load_staged_rhs=0)
        push(1, 0, 0, mxu)
        push(0, 1, 1, mxu)

    def gates_f(rf):
        xpf = xp_scr[pl.ds(rf, Bp), :]
        return (pltpu.matmul_pop(0, (Bp, 256), f32, 0) + xpf[:, 0:256],
                pltpu.matmul_pop(8, (Bp, 256), f32, 0) + xpf[:, 256:512],
                pltpu.matmul_pop(0, (Bp, 256), f32, 1) + xpf[:, 512:768],
                pltpu.matmul_pop(8, (Bp, 256), f32, 1) + xpf[:, 768:1024])

    def gates_b(rb):
        xpb = xp_scr[pl.ds(rb, Bp), :]
        return (pltpu.matmul_pop(16, (Bp, 256), f32, 0) + xpb[:, 1024:1280],
                pltpu.matmul_pop(24, (Bp, 256), f32, 0) + xpb[:, 1280:1536],
                pltpu.matmul_pop(16, (Bp, 256), f32, 1) + xpb[:, 1536:1792],
                pltpu.matmul_pop(24, (Bp, 256), f32, 1) + xpb[:, 1792:2048])

    def step_even(s0, hf, cf, hb, cb):
        rf = pl.multiple_of(s0 * Bp, Bp)
        rb = pl.multiple_of((T - 1 - s0) * Bp, Bp)
        hf16 = hf.astype(bf16)
        hb16 = hb.astype(bf16)
        for mxu in range(2):
            pltpu.matmul_acc_lhs(0, hf16, mxu, load_staged_rhs=None)
            pltpu.matmul_acc_lhs(8, hf16, mxu, load_staged_rhs=1)
            pltpu.matmul_acc_lhs(16, hb16, mxu, load_staged_rhs=0)
            push(1, 1, 1, mxu)
            pltpu.matmul_acc_lhs(24, hb16, mxu, load_staged_rhs=1)
            push(1, 0, 0, mxu)
            push(0, 1, 1, mxu)
        hf, cf = _cell(*gates_f(rf), cf)
        store_fwd(rf, s0, hf)
        hb, cb = _cell(*gates_b(rb), cb)
        store_bwd(rb, T - 1 - s0, hb)
        return hf, cf, hb, cb

    def step_odd(s1, hf, cf, hb, cb):
        rf = pl.multiple_of(s1 * Bp, Bp)
        rb = pl.multiple_of((T - 1 - s1) * Bp, Bp)
        hf16 = hf.astype(bf16)
        hb16 = hb.astype(bf16)
        for mxu in range(2):
            pltpu.matmul_acc_lhs(24, hb16, mxu, load_staged_rhs=None)
            pltpu.matmul_acc_lhs(16, hb16, mxu, load_staged_rhs=0)
            pltpu.matmul_acc_lhs(8, hf16, mxu, load_staged_rhs=1)
            push(0, 0, 0, mxu)
            pltpu.matmul_acc_lhs(0, hf16, mxu, load_staged_rhs=0)
            push(1, 0, 0, mxu)
            push(0, 1, 1, mxu)
        hb, cb = _cell(*gates_b(rb), cb)
        store_bwd(rb, T - 1 - s1, hb)
        hf, cf = _cell(*gates_f(rf), cf)
        store_fwd(rf, s1, hf)
        return hf, cf, hb, cb

    def body(p4, carry):
        s0 = 4 * p4
        carry = step_even(s0, *carry)
        carry = step_odd(s0 + 1, *carry)
        carry = step_even(s0 + 2, *carry)
        carry = step_odd(s0 + 3, *carry)
        return carry

    lax.fori_loop(0, T // 4, body, (z, z, z, z))
    for mxu in range(2):
        pltpu.matmul_acc_lhs(32, z16, mxu, load_staged_rhs=0)
        pltpu.matmul_acc_lhs(32, z16, mxu, load_staged_rhs=1)
        pltpu.matmul_pop(32, (Bp, H), jnp.float32, mxu)


def _fused_kernel(sp_ref, cw_ref, cb_ref,
                  wih1_hbm, b1_ref, whh1f_hbm, whh1b_hbm,
                  wih2f_hbm, wih2b_hbm, b2_ref, whh2f_hbm, whh2b_hbm,
                  fc1wf_hbm, fc1wb_hbm, fc1b_ref,
                  fc2w_ref, fc2b_ref, fc3w_ref, fc3b_ref,
                  o_ref,
                  pscr, cwscr, feat_scr, xp_scr, whh16_scr,
                  l1_scr, h2f_scr, h2b_scr, fc1wf_scr, fc1wb_scr,
                  wih1_ref, whh1f_ref, whh1b_ref, wih2f_ref, wih2b_ref,
                  whh2f_ref, whh2b_ref,
                  semf, semb, sem1, semh1, sem2, semh2, *, Bp, H, FCH):
    f32 = jnp.float32
    CK2 = sp_ref.shape[1]
    K2 = CK2 // 4

    pltpu.make_async_copy(wih1_hbm, wih1_ref, sem1).start()
    pltpu.make_async_copy(whh1f_hbm, whh1f_ref, semh1).start()
    pltpu.make_async_copy(whh1b_hbm, whh1b_ref, semh1).start()
    pltpu.make_async_copy(wih2f_hbm, wih2f_ref, sem2).start()
    pltpu.make_async_copy(wih2b_hbm, wih2b_ref, sem2).start()
    pltpu.make_async_copy(whh2f_hbm, whh2f_ref, semh2).start()
    pltpu.make_async_copy(whh2b_hbm, whh2b_ref, semh2).start()
    pltpu.make_async_copy(fc1wf_hbm, fc1wf_scr, semf).start()
    pltpu.make_async_copy(fc1wb_hbm, fc1wb_scr, semb).start()

    pscr[...] = jnp.zeros((T * Bp, 256), bf16)
    pscr[:, pl.ds(0, CK2)] = sp_ref[...]

    for p in range(POOL):
        cwscr[...] = jnp.zeros((256, 256), f32)
        r0 = 0
        for bi, K in enumerate(CONV_KS):
            off = p + (CONV_KS[-1] - 1) // 2 - (K - 1) // 2
            for c in range(4):
                rows = pl.ds(c * K2 + off, K)
                src = cw_ref[pl.ds(r0 + c * K, K), :]
                if bi == 0:
                    cwscr[rows, :] = src
                else:
                    cwscr[rows, :] = cwscr[rows, :] + src
            r0 += 4 * K
        mxu = p % 2
        pltpu.matmul_push_rhs(cwscr[...].astype(bf16), 0, mxu)
        _mm576(pscr, 0, mxu, 0)
        for mc in range(0, T * Bp, MC):
            v = pltpu.matmul_pop(mc // 4, (MC, 256), f32, mxu)
            if p == 0:
                feat_scr[pl.ds(mc, MC), :] = v
            elif p == 1:
                feat_scr[pl.ds(mc, MC), :] = jnp.maximum(
                    feat_scr[pl.ds(mc, MC), :], v)
            else:
                feat_scr[pl.ds(mc, MC), :] = jnp.maximum(
                    jnp.maximum(feat_scr[pl.ds(mc, MC), :], v) + cb_ref[...],
                    0.0)

    pltpu.make_async_copy(wih1_hbm, wih1_ref, sem1).wait()
    for n in range(8):
        mxu = n % 2
        pltpu.matmul_push_rhs(
            wih1_ref[:, pl.ds(n * 256, 256)].astype(bf16), 0, mxu)
        _mm576(feat_scr, 0, mxu, 0)
        _pop576(xp_scr, n * 256, mxu, b1_ref[0, pl.ds(n * 256, 256)][None, :])

    pltpu.make_async_copy(whh1f_hbm, whh1f_ref, semh1).wait()
    pltpu.make_async_copy(whh1b_hbm, whh1b_ref, semh1).wait()
    whh16_scr[0] = whh1f_ref[...].astype(bf16)
    whh16_scr[1] = whh1b_ref[...].astype(bf16)

    def store_fwd1(r, t, h):
        l1_scr[0, pl.ds(r, Bp), :] = h

    def store_bwd1(r, t, h):
        l1_scr[1, pl.ds(r, Bp), :] = h

    _lstm_bidir_loop(xp_scr, whh16_scr, store_fwd1, store_bwd1, Bp, H)

    pltpu.make_async_copy(wih2f_hbm, wih2f_ref, sem2).wait()
    pltpu.make_async_copy(wih2b_hbm, wih2b_ref, sem2).wait()
    for n in range(8):
        mxu = n % 2
        pltpu.matmul_push_rhs(
            wih2f_ref[:, pl.ds(n * 256, 256)].astype(bf16), 0, mxu)
        pltpu.matmul_push_rhs(
            wih2b_ref[:, pl.ds(n * 256, 256)].astype(bf16), 1, mxu)
        _mm576(l1_scr.at[0], 0, mxu, 0)
        _mm576(l1_scr.at[1], 0, mxu, 1)
        _pop576(xp_scr, n * 256, mxu, b2_ref[0, pl.ds(n * 256, 256)][None, :])

    pltpu.make_async_copy(whh2f_hbm, whh2f_ref, semh2).wait()
    pltpu.make_async_copy(whh2b_hbm, whh2b_ref, semh2).wait()
    whh16_scr[0] = whh2f_ref[...].astype(bf16)
    whh16_scr[1] = whh2b_ref[...].astype(bf16)

    def store_fwd(r, t, h):
        h2f_scr[:, pl.ds(pl.multiple_of(t * H, H), H)] = h

    def store_bwd(r, t, h):
        h2b_scr[:, pl.ds(pl.multiple_of(t * H, H), H)] = h

    _lstm_bidir_loop(xp_scr, whh16_scr, store_fwd, store_bwd, Bp, H)

    pltpu.make_async_copy(fc1wf_hbm, fc1wf_scr, semf).wait()
    pltpu.make_async_copy(fc1wb_hbm, fc1wb_scr, semb).wait()
    for kt in range(T):
        msr = kt % 2
        pltpu.matmul_push_rhs(
            fc1wf_scr[pl.ds(kt * 256, 256), :].astype(bf16), msr, 0)
        pltpu.matmul_acc_lhs(0, h2f_scr[:, pl.ds(kt * 256, 256)].astype(bf16),
                             0, load_staged_rhs=msr)
        pltpu.matmul_push_rhs(
            fc1wb_scr[pl.ds(kt * 256, 256), :].astype(bf16), msr, 1)
        pltpu.matmul_acc_lhs(0, h2b_scr[:, pl.ds(kt * 256, 256)].astype(bf16),
                             1, load_staged_rhs=msr)
    acc = (pltpu.matmul_pop(0, (Bp, FCH), f32, 0)
           + pltpu.matmul_pop(0, (Bp, FCH), f32, 1))

    y = jnp.maximum(acc + fc1b_ref[...], 0.0)
    pltpu.matmul_push_rhs(fc2w_ref[...].astype(bf16), 0, 0)
    pltpu.matmul_acc_lhs(0, y.astype(bf16), 0, load_staged_rhs=0)
    y = jnp.maximum(pltpu.matmul_pop(0, (Bp, FCH), f32, 0)
                    + fc2b_ref[...], 0.0)
    o_ref[...] = jnp.sum(y * fc3w_ref[...], axis=1, keepdims=True) + fc3b_ref[...]


def kernel(x, cw, cb, wih1, b1, whh1f, whh1b, wih2f, wih2b, b2, whh2f, whh2b,
           fc1wf, fc1wb, fc1b, fc2w, fc2b, fc3w, fc3b):
    f32 = jnp.float32
    B, L, Cin = x.shape
    H = whh1f.shape[0]
    FCH = fc2w.shape[0]
    C = cw.shape[1]
    Bp = max(8, (B + 7) // 8 * 8)

    xb = jnp.pad(x.astype(f32), ((0, Bp - B), (0, 0), (0, 0)))
    x_bcl = jnp.transpose(xb, (0, 2, 1))

    Kmax = max(CONV_KS)
    K2 = Kmax + POOL
    pad_big = (Kmax - 1) // 2
    NW = K2 // POOL
    Lp = POOL * (T + NW - 1)
    xpd = jnp.pad(x_bcl.astype(bf16),
                  ((0, 0), (0, 0), (pad_big, Lp - L - pad_big)))
    xt3 = jnp.transpose(xpd, (2, 0, 1)).reshape(Lp // POOL, POOL * Bp * Cin)
    slabs = jnp.concatenate([xt3[w:w + T] for w in range(NW)], axis=1)
    spatch = jnp.transpose(slabs.reshape(T, NW, POOL, Bp, Cin),
                           (0, 3, 4, 1, 2)).reshape(T * Bp, Cin * K2)
    CK2 = Cin * K2

    out = pl.pallas_call(
        functools.partial(_fused_kernel, Bp=Bp, H=H, FCH=FCH),
        out_shape=jax.ShapeDtypeStruct((Bp, 1), f32),
        in_specs=[
            _full((T * Bp, CK2)),
            _full((cw.shape[0], C)), _full((1, C)),
            pl.BlockSpec(memory_space=pl.ANY), _full((1, 8 * H)),
            pl.BlockSpec(memory_space=pl.ANY),
            pl.BlockSpec(memory_space=pl.ANY),
            pl.BlockSpec(memory_space=pl.ANY),
            pl.BlockSpec(memory_space=pl.ANY), _full((1, 8 * H)),
            pl.BlockSpec(memory_space=pl.ANY),
            pl.BlockSpec(memory_space=pl.ANY),
            pl.BlockSpec(memory_space=pl.ANY),
            pl.BlockSpec(memory_space=pl.ANY),
            _full((1, FCH)),
            _full((FCH, FCH)), _full((1, FCH)),
            _full((1, FCH)), _full((1, 1)),
        ],
        out_specs=_full((Bp, 1)),
        scratch_shapes=[
            pltpu.VMEM((T * Bp, 256), bf16),
            pltpu.VMEM((256, 256), f32),
            pltpu.VMEM((T * Bp, C), f32),
            pltpu.VMEM((T * Bp, 8 * H), f32),
            pltpu.VMEM((2, H, 4 * H), bf16),
            pltpu.VMEM((2, T * Bp, H), f32),
            pltpu.VMEM((Bp, T * H), f32),
            pltpu.VMEM((Bp, T * H), f32),
            pltpu.VMEM((T * H, FCH), f32),
            pltpu.VMEM((T * H, FCH), f32),
            pltpu.VMEM((C, 8 * H), f32),
            pltpu.VMEM((H, 4 * H), f32),
            pltpu.VMEM((H, 4 * H), f32),
            pltpu.VMEM((H, 8 * H), f32),
            pltpu.VMEM((H, 8 * H), f32),
            pltpu.VMEM((H, 4 * H), f32),
            pltpu.VMEM((H, 4 * H), f32),
            pltpu.SemaphoreType.DMA,
            pltpu.SemaphoreType.DMA,
            pltpu.SemaphoreType.DMA,
            pltpu.SemaphoreType.DMA,
            pltpu.SemaphoreType.DMA,
            pltpu.SemaphoreType.DMA,
        ],
        grid=(),
    )(spatch, cw, cb, wih1, b1, whh1f, whh1b,
      wih2f, wih2b, b2, whh2f, whh2b, fc1wf, fc1wb, fc1b,
      fc2w, fc2b, fc3w, fc3b)

    return out[:B, 0]
